# baseline (speedup 1.0000x reference)
import functools

import jax
import jax.numpy as jnp
from jax import lax
from jax.experimental import pallas as pl
from jax.experimental.pallas import tpu as pltpu
from jax.experimental.pallas import tpu_sc as plsc

N = 10000
E = 320000
D = 128
HOP = 5
NC = 2
NS = 16
L = 16

EC = 80
EPT = E // NS
NCHUNK_E = EPT // EC
BLKC = 8
NBLKT = (E // EC) // BLKC
NB = 16
NCHUNK_N = N // NB


def _leaky_v(v):
    return jnp.where(v > 0, v, 0.2 * v)


_BLK = 1000


def _prologue_body(xa_ref, xb_ref, w1a_ref, b1a_ref, w1b_ref, b1b_ref,
                   a1ab_ref, a2ab_ref, a1ba_ref, a2ba_ref,
                   xA_out, xB_out, x1ab_out, w2ab_out, x1ba_out, w2ba_out,
                   h1b0_out, h1a0_out):
    xa = jnp.maximum(
        jnp.dot(xa_ref[...], w1a_ref[...], preferred_element_type=jnp.float32)
        + b1a_ref[...], 0.0)
    xb = jnp.maximum(
        jnp.dot(xb_ref[...], w1b_ref[...], preferred_element_type=jnp.float32)
        + b1b_ref[...], 0.0)
    xA_out[...] = xa
    xB_out[...] = xb
    dn = (((1,), (1,)), ((), ()))
    p1 = lax.dot_general(xa, a1ab_ref[...], dn, preferred_element_type=jnp.float32)
    p2 = lax.dot_general(xa, a2ab_ref[...], dn, preferred_element_type=jnp.float32)
    x1ab_out[...] = p1
    w2ab_out[...] = jnp.exp(_leaky_v(p1 + p2))
    q1 = lax.dot_general(xb, a1ba_ref[...], dn, preferred_element_type=jnp.float32)
    q2 = lax.dot_general(xb, a2ba_ref[...], dn, preferred_element_type=jnp.float32)
    x1ba_out[...] = q1
    w2ba_out[...] = jnp.exp(_leaky_v(q1 + q2))
    h1b0_out[...] = lax.dot_general(xb, a2ab_ref[0:1, :], dn,
                                    preferred_element_type=jnp.float32)
    h1a0_out[...] = lax.dot_general(xa, a2ba_ref[0:1, :], dn,
                                    preferred_element_type=jnp.float32)


def _run_prologue(x_A, x_B, W1_A, b1_A, W1_B, b1_B, a1_ab, a2_ab, a1_ba, a2_ba):
    grid = (N // _BLK,)
    full = lambda shp: pl.BlockSpec(shp, lambda i: (0, 0))
    rows = pl.BlockSpec((_BLK, D), lambda i: (i, 0))
    cols5 = pl.BlockSpec((_BLK, HOP), lambda i: (i, 0))
    cols1 = pl.BlockSpec((_BLK, 1), lambda i: (i, 0))
    out_shape = [
        jax.ShapeDtypeStruct((N, D), jnp.float32),
        jax.ShapeDtypeStruct((N, D), jnp.float32),
        jax.ShapeDtypeStruct((N, HOP), jnp.float32),
        jax.ShapeDtypeStruct((N, HOP), jnp.float32),
        jax.ShapeDtypeStruct((N, HOP), jnp.float32),
        jax.ShapeDtypeStruct((N, HOP), jnp.float32),
        jax.ShapeDtypeStruct((N, 1), jnp.float32),
        jax.ShapeDtypeStruct((N, 1), jnp.float32),
    ]
    return pl.pallas_call(
        _prologue_body,
        grid=grid,
        in_specs=[rows, rows, full((D, D)), full((1, D)), full((D, D)),
                  full((1, D)), full((HOP, D)), full((HOP, D)),
                  full((HOP, D)), full((HOP, D))],
        out_specs=[rows, rows, cols5, cols5, cols5, cols5, cols1, cols1],
        out_shape=out_shape,
    )(x_A, x_B, W1_A, b1_A.reshape(1, D), W1_B, b1_B.reshape(1, D),
      a1_ab, a2_ab, a1_ba, a2_ba)


def _epilogue_body(h_ref, w2_ref, b2_ref, out_ref):
    out_ref[...] = (
        jnp.dot(h_ref[...], w2_ref[...], preferred_element_type=jnp.float32)
        + b2_ref[...])


def _run_epilogue(hA, W2, b2):
    n_out = W2.shape[1]
    return pl.pallas_call(
        _epilogue_body,
        grid=(N // _BLK,),
        in_specs=[pl.BlockSpec((_BLK, D), lambda i: (i, 0)),
                  pl.BlockSpec((D, n_out), lambda i: (0, 0)),
                  pl.BlockSpec((1, n_out), lambda i: (0, 0))],
        out_specs=pl.BlockSpec((_BLK, n_out), lambda i: (i, 0)),
        out_shape=jax.ShapeDtypeStruct((N, n_out), jnp.float32),
    )(hA, W2, b2.reshape(1, n_out))


def _hop_body(hA, hB, xA, xB, x1ab, w2ab, x1ba, w2ba, h1B, h1A,
              sab, tab, sba, tba,
              a2abn, a2ban,
              newhA, newhB, h1An, h1Bn,
              x1tab, h1tab, a2buf, sidx, tidx, rows, w1buf,
              aggbuf, outbuf, divbuf, w2buf, h1nbuf,
              agg_sp, div_sp, sem0, sem1, ssem0, ssem1, dsem0, dsem1):
    c = lax.axis_index("c")
    s = lax.axis_index("s")
    sems = (sem0, sem1)
    ssems = (ssem0, ssem1)
    dsems = (dsem0, dsem1)

    def pipeline(h_ref, x_ref, x1_ref, w2_ref, h1_ref, es_ref, et_ref,
                 a2n_ref, newh_ref, h1n_ref):
        pltpu.sync_copy(x1_ref, x1tab)
        pltpu.sync_copy(h1_ref, h1tab)
        pltpu.sync_copy(a2n_ref, a2buf)

        nck = (NCHUNK_N - s + NS - 1) // NS

        def ichunk(k, _):
            ci = s + k * NS
            nb = ci * NB
            pltpu.sync_copy(x_ref.at[pl.ds(nb, NB), :], aggbuf)
            pltpu.sync_copy(w2_ref.at[pl.ds(nb, NB)], w2buf)
            w2v = w2buf[...]
            for e in range(NB):
                w2 = w2v[e]
                for f in range(D // L):
                    outbuf[e, pl.ds(f * L, L)] = (
                        aggbuf[e, pl.ds(f * L, L)] * w2)
            pltpu.sync_copy(outbuf, agg_sp.at[pl.ds(nb, NB), :])
            pltpu.sync_copy(w2buf, div_sp.at[pl.ds(nb, NB)])
            return 0
        lax.fori_loop(0, nck, ichunk, 0)
        plsc.subcore_barrier()


        def compute_w1(j, b):
            def wgroup(g, _):
                gb = g * L
                t16 = tidx[j, pl.ds(gb, L)]
                s16 = sidx[j, pl.ds(gb, L)]
                h1t = plsc.load_gather(h1tab, [t16])
                x1s = plsc.load_gather(x1tab, [s16])
                w1buf[b, pl.ds(gb, L)] = jnp.exp(_leaky_v(x1s + h1t))
                return 0
            lax.fori_loop(0, EC // L, wgroup, 0)

        def issue_gather(j, b):
            pltpu.async_copy(h_ref.at[tidx.at[j]], rows.at[b], sems[b])

        def process(j, b):
            pltpu.make_async_copy(h_ref.at[tidx.at[j]], rows.at[b],
                                  sems[b]).wait()

            def egroup(g, _):
                gb = g * L
                w16 = w1buf[b, pl.ds(gb, L)]
                for e in range(L):
                    w = w16[e]
                    r = gb + e
                    for f in range(D // L):
                        rows[b, r, pl.ds(f * L, L)] = (
                            rows[b, r, pl.ds(f * L, L)] * w)
                return 0
            lax.fori_loop(0, EC // L, egroup, 0)
            pltpu.async_copy(rows.at[b], agg_sp.at[sidx.at[j]], ssems[b],
                             add=True)
            pltpu.async_copy(w1buf.at[b], div_sp.at[sidx.at[j]], dsems[b],
                             add=True)

        def bloop(blk, _):
            rowbase = (s + blk * NS) * BLKC
            pltpu.sync_copy(es_ref.at[pl.ds(rowbase, BLKC), :], sidx)
            pltpu.sync_copy(et_ref.at[pl.ds(rowbase, BLKC), :], tidx)
            issue_gather(0, 0)
            compute_w1(0, 0)

            def jpair(jp, _):
                for b in (0, 1):
                    j = jp * 2 + b

                    @pl.when(j + 1 < BLKC)
                    def _():
                        @pl.when(j >= 1)
                        def _():
                            pltpu.make_async_copy(
                                rows.at[1 - b], agg_sp.at[sidx.at[0]],
                                ssems[1 - b]).wait()
                            pltpu.make_async_copy(
                                w1buf.at[1 - b], div_sp.at[sidx.at[0]],
                                dsems[1 - b]).wait()
                        issue_gather(j + 1, 1 - b)
                        compute_w1(j + 1, 1 - b)

                    process(j, b)
                return 0
            lax.fori_loop(0, BLKC // 2, jpair, 0)
            for b in (0, 1):
                pltpu.make_async_copy(rows.at[b], agg_sp.at[sidx.at[0]],
                                      ssems[b]).wait()
                pltpu.make_async_copy(w1buf.at[b], div_sp.at[sidx.at[0]],
                                      dsems[b]).wait()
            return 0
        nbe = (NBLKT - s + NS - 1) // NS
        lax.fori_loop(0, nbe, bloop, 0)
        plsc.subcore_barrier()

        def nchunk(k, _):
            ci = s + k * NS
            nb = ci * NB
            pltpu.sync_copy(agg_sp.at[pl.ds(nb, NB), :], aggbuf)
            pltpu.sync_copy(div_sp.at[pl.ds(nb, NB)], divbuf)
            dvv = divbuf[...]
            lane = lax.iota(jnp.int32, L)
            hv = jnp.zeros((L,), jnp.float32)
            for e in range(L):
                dv = dvv[e]
                acc = jnp.zeros((L,), jnp.float32)
                for f in range(D // L):
                    v = aggbuf[e, pl.ds(f * L, L)] / dv
                    nh = jnp.where(v > 0, v, jnp.exp(v) - 1.0)
                    outbuf[e, pl.ds(f * L, L)] = nh
                    acc = acc + nh * a2buf[pl.ds(f * L, L)]
                hv = jnp.where(lane == e, jnp.sum(acc), hv)
            h1nbuf[...] = hv
            pltpu.sync_copy(outbuf, newh_ref.at[pl.ds(nb, NB), :])
            pltpu.sync_copy(h1nbuf, h1n_ref.at[pl.ds(nb, NB)])
            return 0
        lax.fori_loop(0, nck, nchunk, 0)

    @pl.when(c == 0)
    def _():
        pipeline(hB, xA, x1ab, w2ab, h1B, sab, tab, a2ban, newhA, h1An)

    @pl.when(c == 1)
    def _():
        pipeline(hA, xB, x1ba, w2ba, h1A, sba, tba, a2abn, newhB, h1Bn)


def _run_hop(hA, hB, xA, xB, x1ab_i, w2ab_i, x1ba_i, w2ba_i, h1B, h1A,
             sab, tab, sba, tba, a2ab_next, a2ba_next):
    mesh = plsc.VectorSubcoreMesh(core_axis_name="c", subcore_axis_name="s",
                                  num_cores=NC, num_subcores=NS)
    f32 = jnp.float32
    i32 = jnp.int32
    out_type = [
        jax.ShapeDtypeStruct((N, D), f32),
        jax.ShapeDtypeStruct((N, D), f32),
        jax.ShapeDtypeStruct((N,), f32),
        jax.ShapeDtypeStruct((N,), f32),
    ]
    scratch = [
        pltpu.VMEM((N,), f32),
        pltpu.VMEM((N,), f32),
        pltpu.VMEM((D,), f32),
        pltpu.VMEM((BLKC, EC), i32),
        pltpu.VMEM((BLKC, EC), i32),
        pltpu.VMEM((2, EC, D), f32),
        pltpu.VMEM((2, EC), f32),
        pltpu.VMEM((NB, D), f32),
        pltpu.VMEM((NB, D), f32),
        pltpu.VMEM((NB,), f32),
        pltpu.VMEM((NB,), f32),
        pltpu.VMEM((NB,), f32),
        pltpu.VMEM_SHARED((N, D), f32),
        pltpu.VMEM_SHARED((N,), f32),
        pltpu.SemaphoreType.DMA,
        pltpu.SemaphoreType.DMA,
        pltpu.SemaphoreType.DMA,
        pltpu.SemaphoreType.DMA,
        pltpu.SemaphoreType.DMA,
        pltpu.SemaphoreType.DMA,
    ]
    run = pl.kernel(_hop_body, out_type=out_type, mesh=mesh,
                    scratch_types=scratch,
                    compiler_params=pltpu.CompilerParams(
                        needs_layout_passes=False))
    return run(hA, hB, xA, xB, x1ab_i, w2ab_i, x1ba_i, w2ba_i, h1B, h1A,
               sab, tab, sba, tba, a2ab_next, a2ba_next)


def kernel(x_A, x_B, edge_ab, edge_ba, W1_A, b1_A, W1_B, b1_B,
           a1_ab, a2_ab, a1_ba, a2_ba, W2, b2):
    (xA, xB, x1ab, w2ab, x1ba, w2ba, h1b0, h1a0) = _run_prologue(
        x_A, x_B, W1_A, b1_A, W1_B, b1_B, a1_ab, a2_ab, a1_ba, a2_ba)
    hA, hB = xA, xB
    h1B, h1A = h1b0.reshape(N), h1a0.reshape(N)
    sab, tab = edge_ab[0].reshape(E // EC, EC), edge_ab[1].reshape(E // EC, EC)
    sba, tba = edge_ba[0].reshape(E // EC, EC), edge_ba[1].reshape(E // EC, EC)
    for i in range(HOP):
        nxt = i + 1 if i + 1 < HOP else 0
        hA, hB, h1A, h1B = _run_hop(
            hA, hB, xA, xB, x1ab[:, i], w2ab[:, i], x1ba[:, i], w2ba[:, i],
            h1B, h1A, sab, tab, sba, tba, a2_ab[nxt], a2_ba[nxt])
    return _run_epilogue(hA, W2, b2)


# speedup vs baseline: 1.0594x; 1.0594x over previous
import functools

import jax
import jax.numpy as jnp
from jax import lax
from jax.experimental import pallas as pl
from jax.experimental.pallas import tpu as pltpu
from jax.experimental.pallas import tpu_sc as plsc

N = 10000
E = 320000
D = 128
HOP = 5
NC = 2
NS = 16
L = 16

EC = 80
EPT = E // NS
NCHUNK_E = EPT // EC
BLKC = 8
NBLKT = (E // EC) // BLKC
NB2 = 32
NSTEP = (N + NB2 - 1) // NB2


def _leaky_v(v):
    return jnp.where(v > 0, v, 0.2 * v)


_BLK = 1000


def _prologue_body(xa_ref, xb_ref, w1a_ref, b1a_ref, w1b_ref, b1b_ref,
                   a1ab_ref, a2ab_ref, a1ba_ref, a2ba_ref,
                   xA_out, xB_out, x1ab_out, w2ab_out, x1ba_out, w2ba_out,
                   h1b0_out, h1a0_out):
    xa = jnp.maximum(
        jnp.dot(xa_ref[...], w1a_ref[...], preferred_element_type=jnp.float32)
        + b1a_ref[...], 0.0)
    xb = jnp.maximum(
        jnp.dot(xb_ref[...], w1b_ref[...], preferred_element_type=jnp.float32)
        + b1b_ref[...], 0.0)
    xA_out[...] = xa
    xB_out[...] = xb
    dn = (((1,), (1,)), ((), ()))
    p1 = lax.dot_general(xa, a1ab_ref[...], dn, preferred_element_type=jnp.float32)
    p2 = lax.dot_general(xa, a2ab_ref[...], dn, preferred_element_type=jnp.float32)
    x1ab_out[...] = p1
    w2ab_out[...] = jnp.exp(_leaky_v(p1 + p2))
    q1 = lax.dot_general(xb, a1ba_ref[...], dn, preferred_element_type=jnp.float32)
    q2 = lax.dot_general(xb, a2ba_ref[...], dn, preferred_element_type=jnp.float32)
    x1ba_out[...] = q1
    w2ba_out[...] = jnp.exp(_leaky_v(q1 + q2))
    h1b0_out[...] = lax.dot_general(xb, a2ab_ref[0:1, :], dn,
                                    preferred_element_type=jnp.float32)
    h1a0_out[...] = lax.dot_general(xa, a2ba_ref[0:1, :], dn,
                                    preferred_element_type=jnp.float32)


def _run_prologue(x_A, x_B, W1_A, b1_A, W1_B, b1_B, a1_ab, a2_ab, a1_ba, a2_ba):
    grid = (N // _BLK,)
    full = lambda shp: pl.BlockSpec(shp, lambda i: (0, 0))
    rows = pl.BlockSpec((_BLK, D), lambda i: (i, 0))
    cols5 = pl.BlockSpec((_BLK, HOP), lambda i: (i, 0))
    cols1 = pl.BlockSpec((_BLK, 1), lambda i: (i, 0))
    out_shape = [
        jax.ShapeDtypeStruct((N, D), jnp.float32),
        jax.ShapeDtypeStruct((N, D), jnp.float32),
        jax.ShapeDtypeStruct((N, HOP), jnp.float32),
        jax.ShapeDtypeStruct((N, HOP), jnp.float32),
        jax.ShapeDtypeStruct((N, HOP), jnp.float32),
        jax.ShapeDtypeStruct((N, HOP), jnp.float32),
        jax.ShapeDtypeStruct((N, 1), jnp.float32),
        jax.ShapeDtypeStruct((N, 1), jnp.float32),
    ]
    return pl.pallas_call(
        _prologue_body,
        grid=grid,
        in_specs=[rows, rows, full((D, D)), full((1, D)), full((D, D)),
                  full((1, D)), full((HOP, D)), full((HOP, D)),
                  full((HOP, D)), full((HOP, D))],
        out_specs=[rows, rows, cols5, cols5, cols5, cols5, cols1, cols1],
        out_shape=out_shape,
    )(x_A, x_B, W1_A, b1_A.reshape(1, D), W1_B, b1_B.reshape(1, D),
      a1_ab, a2_ab, a1_ba, a2_ba)


def _epilogue_body(h_ref, w2_ref, b2_ref, out_ref):
    out_ref[...] = (
        jnp.dot(h_ref[...], w2_ref[...], preferred_element_type=jnp.float32)
        + b2_ref[...])


def _run_epilogue(hA, W2, b2):
    n_out = W2.shape[1]
    return pl.pallas_call(
        _epilogue_body,
        grid=(N // _BLK,),
        in_specs=[pl.BlockSpec((_BLK, D), lambda i: (i, 0)),
                  pl.BlockSpec((D, n_out), lambda i: (0, 0)),
                  pl.BlockSpec((1, n_out), lambda i: (0, 0))],
        out_specs=pl.BlockSpec((_BLK, n_out), lambda i: (i, 0)),
        out_shape=jax.ShapeDtypeStruct((N, n_out), jnp.float32),
    )(hA, W2, b2.reshape(1, n_out))


def _hop_body(hA, hB, xA, xB, x1ab, w2ab, x1ba, w2ba, h1B, h1A,
              sab, tab, sba, tba,
              a2abn, a2ban,
              newhA, newhB, h1An, h1Bn,
              x1tab, h1tab, a2buf, sidx, tidx, rows, w1buf,
              nbuf, divbuf, w2buf, h1nbuf,
              agg_sp, div_sp, sem0, sem1, ssem0, ssem1, dsem0, dsem1):
    c = lax.axis_index("c")
    s = lax.axis_index("s")
    sems = (sem0, sem1)
    ssems = (ssem0, ssem1)
    dsems = (dsem0, dsem1)

    def pipeline(h_ref, x_ref, x1_ref, w2_ref, h1_ref, es_ref, et_ref,
                 a2n_ref, newh_ref, h1n_ref):
        pltpu.sync_copy(x1_ref, x1tab)
        pltpu.sync_copy(h1_ref, h1tab)
        pltpu.sync_copy(a2n_ref, a2buf)

        nst = jnp.where(s < NSTEP - 19 * NS, 20, 19)
        lo = jnp.where(s < NSTEP - 19 * NS, 20 * s,
                       19 * s + (NSTEP - 19 * NS))

        def _step_base(k):
            return jnp.minimum((lo + k) * NB2, N - NB2)

        def ichunk(k, _):
            nb = _step_base(k)
            pltpu.sync_copy(x_ref.at[pl.ds(nb, NB2), :], nbuf)
            pltpu.sync_copy(w2_ref.at[pl.ds(nb, NB2)], w2buf)

            def ihalf(half, _):
                w2v = w2buf[pl.ds(half * L, L)]
                for e in range(L):
                    w2 = w2v[e]
                    r = half * L + e
                    for f in range(D // L):
                        nbuf[r, pl.ds(f * L, L)] = (
                            nbuf[r, pl.ds(f * L, L)] * w2)
                return 0
            lax.fori_loop(0, 2, ihalf, 0)
            pltpu.sync_copy(nbuf, agg_sp.at[pl.ds(nb, NB2), :])
            pltpu.sync_copy(w2buf, div_sp.at[pl.ds(nb, NB2)])
            return 0
        lax.fori_loop(0, nst, ichunk, 0)
        plsc.subcore_barrier()


        def compute_w1(j, b):
            def wgroup(g, _):
                gb = g * L
                t16 = tidx[j, pl.ds(gb, L)]
                s16 = sidx[j, pl.ds(gb, L)]
                h1t = plsc.load_gather(h1tab, [t16])
                x1s = plsc.load_gather(x1tab, [s16])
                w1buf[b, pl.ds(gb, L)] = jnp.exp(_leaky_v(x1s + h1t))
                return 0
            lax.fori_loop(0, EC // L, wgroup, 0)

        def issue_gather(j, b):
            pltpu.async_copy(h_ref.at[tidx.at[j]], rows.at[b], sems[b])

        def process(j, b):
            pltpu.make_async_copy(h_ref.at[tidx.at[j]], rows.at[b],
                                  sems[b]).wait()

            def egroup(g, _):
                gb = g * L
                w16 = w1buf[b, pl.ds(gb, L)]
                for e in range(L):
                    w = w16[e]
                    r = gb + e
                    for f in range(D // L):
                        rows[b, r, pl.ds(f * L, L)] = (
                            rows[b, r, pl.ds(f * L, L)] * w)
                return 0
            lax.fori_loop(0, EC // L, egroup, 0)
            pltpu.async_copy(rows.at[b], agg_sp.at[sidx.at[j]], ssems[b],
                             add=True)
            pltpu.async_copy(w1buf.at[b], div_sp.at[sidx.at[j]], dsems[b],
                             add=True)

        def bloop(blk, _):
            rowbase = (s + blk * NS) * BLKC
            pltpu.sync_copy(es_ref.at[pl.ds(rowbase, BLKC), :], sidx)
            pltpu.sync_copy(et_ref.at[pl.ds(rowbase, BLKC), :], tidx)
            issue_gather(0, 0)
            compute_w1(0, 0)

            def jpair(jp, _):
                for b in (0, 1):
                    j = jp * 2 + b

                    @pl.when(j + 1 < BLKC)
                    def _():
                        @pl.when(j >= 1)
                        def _():
                            pltpu.make_async_copy(
                                rows.at[1 - b], agg_sp.at[sidx.at[0]],
                                ssems[1 - b]).wait()
                            pltpu.make_async_copy(
                                w1buf.at[1 - b], div_sp.at[sidx.at[0]],
                                dsems[1 - b]).wait()
                        issue_gather(j + 1, 1 - b)
                        compute_w1(j + 1, 1 - b)

                    process(j, b)
                return 0
            lax.fori_loop(0, BLKC // 2, jpair, 0)
            for b in (0, 1):
                pltpu.make_async_copy(rows.at[b], agg_sp.at[sidx.at[0]],
                                      ssems[b]).wait()
                pltpu.make_async_copy(w1buf.at[b], div_sp.at[sidx.at[0]],
                                      dsems[b]).wait()
            return 0
        nbe = (NBLKT - s + NS - 1) // NS
        lax.fori_loop(0, nbe, bloop, 0)
        plsc.subcore_barrier()

        def nchunk(k, _):
            nb = _step_base(k)
            pltpu.sync_copy(agg_sp.at[pl.ds(nb, NB2), :], nbuf)
            pltpu.sync_copy(div_sp.at[pl.ds(nb, NB2)], divbuf)
            lane = lax.iota(jnp.int32, L)

            def nhalf(half, _):
                dvv = divbuf[pl.ds(half * L, L)]
                hv = jnp.zeros((L,), jnp.float32)
                for e in range(L):
                    dv = dvv[e]
                    r = half * L + e
                    acc = jnp.zeros((L,), jnp.float32)
                    for f in range(D // L):
                        v = nbuf[r, pl.ds(f * L, L)] / dv
                        nh = jnp.where(v > 0, v, jnp.exp(v) - 1.0)
                        nbuf[r, pl.ds(f * L, L)] = nh
                        acc = acc + nh * a2buf[pl.ds(f * L, L)]
                    hv = jnp.where(lane == e, jnp.sum(acc), hv)
                h1nbuf[pl.ds(half * L, L)] = hv
                return 0
            lax.fori_loop(0, 2, nhalf, 0)
            pltpu.sync_copy(nbuf, newh_ref.at[pl.ds(nb, NB2), :])
            pltpu.sync_copy(h1nbuf, h1n_ref.at[pl.ds(nb, NB2)])
            return 0
        lax.fori_loop(0, nst, nchunk, 0)

    @pl.when(c == 0)
    def _():
        pipeline(hB, xA, x1ab, w2ab, h1B, sab, tab, a2ban, newhA, h1An)

    @pl.when(c == 1)
    def _():
        pipeline(hA, xB, x1ba, w2ba, h1A, sba, tba, a2abn, newhB, h1Bn)


def _run_hop(hA, hB, xA, xB, x1ab_i, w2ab_i, x1ba_i, w2ba_i, h1B, h1A,
             sab, tab, sba, tba, a2ab_next, a2ba_next):
    mesh = plsc.VectorSubcoreMesh(core_axis_name="c", subcore_axis_name="s",
                                  num_cores=NC, num_subcores=NS)
    f32 = jnp.float32
    i32 = jnp.int32
    out_type = [
        jax.ShapeDtypeStruct((N, D), f32),
        jax.ShapeDtypeStruct((N, D), f32),
        jax.ShapeDtypeStruct((N,), f32),
        jax.ShapeDtypeStruct((N,), f32),
    ]
    scratch = [
        pltpu.VMEM((N,), f32),
        pltpu.VMEM((N,), f32),
        pltpu.VMEM((D,), f32),
        pltpu.VMEM((BLKC, EC), i32),
        pltpu.VMEM((BLKC, EC), i32),
        pltpu.VMEM((2, EC, D), f32),
        pltpu.VMEM((2, EC), f32),
        pltpu.VMEM((NB2, D), f32),
        pltpu.VMEM((NB2,), f32),
        pltpu.VMEM((NB2,), f32),
        pltpu.VMEM((NB2,), f32),
        pltpu.VMEM_SHARED((N, D), f32),
        pltpu.VMEM_SHARED((N,), f32),
        pltpu.SemaphoreType.DMA,
        pltpu.SemaphoreType.DMA,
        pltpu.SemaphoreType.DMA,
        pltpu.SemaphoreType.DMA,
        pltpu.SemaphoreType.DMA,
        pltpu.SemaphoreType.DMA,
    ]
    run = pl.kernel(_hop_body, out_type=out_type, mesh=mesh,
                    scratch_types=scratch,
                    compiler_params=pltpu.CompilerParams(
                        needs_layout_passes=False))
    return run(hA, hB, xA, xB, x1ab_i, w2ab_i, x1ba_i, w2ba_i, h1B, h1A,
               sab, tab, sba, tba, a2ab_next, a2ba_next)


def kernel(x_A, x_B, edge_ab, edge_ba, W1_A, b1_A, W1_B, b1_B,
           a1_ab, a2_ab, a1_ba, a2_ba, W2, b2):
    (xA, xB, x1ab, w2ab, x1ba, w2ba, h1b0, h1a0) = _run_prologue(
        x_A, x_B, W1_A, b1_A, W1_B, b1_B, a1_ab, a2_ab, a1_ba, a2_ba)
    hA, hB = xA, xB
    h1B, h1A = h1b0.reshape(N), h1a0.reshape(N)
    sab, tab = edge_ab[0].reshape(E // EC, EC), edge_ab[1].reshape(E // EC, EC)
    sba, tba = edge_ba[0].reshape(E // EC, EC), edge_ba[1].reshape(E // EC, EC)
    for i in range(HOP):
        nxt = i + 1 if i + 1 < HOP else 0
        hA, hB, h1A, h1B = _run_hop(
            hA, hB, xA, xB, x1ab[:, i], w2ab[:, i], x1ba[:, i], w2ba[:, i],
            h1B, h1A, sab, tab, sba, tba, a2_ab[nxt], a2_ba[nxt])
    return _run_epilogue(hA, W2, b2)


# speedup vs baseline: 1.0655x; 1.0058x over previous
import functools

import jax
import jax.numpy as jnp
from jax import lax
from jax.experimental import pallas as pl
from jax.experimental.pallas import tpu as pltpu
from jax.experimental.pallas import tpu_sc as plsc

N = 10000
E = 320000
D = 128
HOP = 5
NC = 2
NS = 16
L = 16

EC = 80
EPT = E // NS
NCHUNK_E = EPT // EC
BLKC = 8
NBLKT = (E // EC) // BLKC
NB2 = 32
NSTEP = (N + NB2 - 1) // NB2


def _leaky_v(v):
    return jnp.where(v > 0, v, 0.2 * v)


_BLK = 1000


def _prologue_body(xa_ref, xb_ref, w1a_ref, b1a_ref, w1b_ref, b1b_ref,
                   a1ab_ref, a2ab_ref, a1ba_ref, a2ba_ref,
                   xA_out, xB_out, x1ab_out, w2ab_out, x1ba_out, w2ba_out,
                   h1b0_out, h1a0_out):
    xa = jnp.maximum(
        jnp.dot(xa_ref[...], w1a_ref[...], preferred_element_type=jnp.float32)
        + b1a_ref[...], 0.0)
    xb = jnp.maximum(
        jnp.dot(xb_ref[...], w1b_ref[...], preferred_element_type=jnp.float32)
        + b1b_ref[...], 0.0)
    xA_out[...] = xa
    xB_out[...] = xb
    dn = (((1,), (1,)), ((), ()))
    p1 = lax.dot_general(xa, a1ab_ref[...], dn, preferred_element_type=jnp.float32)
    p2 = lax.dot_general(xa, a2ab_ref[...], dn, preferred_element_type=jnp.float32)
    x1ab_out[...] = p1
    w2ab_out[...] = jnp.exp(_leaky_v(p1 + p2))
    q1 = lax.dot_general(xb, a1ba_ref[...], dn, preferred_element_type=jnp.float32)
    q2 = lax.dot_general(xb, a2ba_ref[...], dn, preferred_element_type=jnp.float32)
    x1ba_out[...] = q1
    w2ba_out[...] = jnp.exp(_leaky_v(q1 + q2))
    h1b0_out[...] = lax.dot_general(xb, a2ab_ref[0:1, :], dn,
                                    preferred_element_type=jnp.float32)
    h1a0_out[...] = lax.dot_general(xa, a2ba_ref[0:1, :], dn,
                                    preferred_element_type=jnp.float32)


def _run_prologue(x_A, x_B, W1_A, b1_A, W1_B, b1_B, a1_ab, a2_ab, a1_ba, a2_ba):
    grid = (N // _BLK,)
    full = lambda shp: pl.BlockSpec(shp, lambda i: (0, 0))
    rows = pl.BlockSpec((_BLK, D), lambda i: (i, 0))
    cols5 = pl.BlockSpec((_BLK, HOP), lambda i: (i, 0))
    cols1 = pl.BlockSpec((_BLK, 1), lambda i: (i, 0))
    out_shape = [
        jax.ShapeDtypeStruct((N, D), jnp.float32),
        jax.ShapeDtypeStruct((N, D), jnp.float32),
        jax.ShapeDtypeStruct((N, HOP), jnp.float32),
        jax.ShapeDtypeStruct((N, HOP), jnp.float32),
        jax.ShapeDtypeStruct((N, HOP), jnp.float32),
        jax.ShapeDtypeStruct((N, HOP), jnp.float32),
        jax.ShapeDtypeStruct((N, 1), jnp.float32),
        jax.ShapeDtypeStruct((N, 1), jnp.float32),
    ]
    return pl.pallas_call(
        _prologue_body,
        grid=grid,
        in_specs=[rows, rows, full((D, D)), full((1, D)), full((D, D)),
                  full((1, D)), full((HOP, D)), full((HOP, D)),
                  full((HOP, D)), full((HOP, D))],
        out_specs=[rows, rows, cols5, cols5, cols5, cols5, cols1, cols1],
        out_shape=out_shape,
    )(x_A, x_B, W1_A, b1_A.reshape(1, D), W1_B, b1_B.reshape(1, D),
      a1_ab, a2_ab, a1_ba, a2_ba)


def _epilogue_body(h_ref, w2_ref, b2_ref, out_ref):
    out_ref[...] = (
        jnp.dot(h_ref[...], w2_ref[...], preferred_element_type=jnp.float32)
        + b2_ref[...])


def _run_epilogue(hA, W2, b2):
    n_out = W2.shape[1]
    return pl.pallas_call(
        _epilogue_body,
        grid=(N // _BLK,),
        in_specs=[pl.BlockSpec((_BLK, D), lambda i: (i, 0)),
                  pl.BlockSpec((D, n_out), lambda i: (0, 0)),
                  pl.BlockSpec((1, n_out), lambda i: (0, 0))],
        out_specs=pl.BlockSpec((_BLK, n_out), lambda i: (i, 0)),
        out_shape=jax.ShapeDtypeStruct((N, n_out), jnp.float32),
    )(hA, W2, b2.reshape(1, n_out))


def _hop_body(hA, hB, xA, xB, x1ab, w2ab, x1ba, w2ba, h1B, h1A,
              sab, tab, sba, tba,
              a2abn, a2ban,
              newhA, newhB, h1An, h1Bn,
              x1tab, h1tab, a2buf, sidx, tidx, rows, w1buf,
              nbuf, divbuf, w2buf, h1nbuf,
              agg_sp, div_sp, sem0, sem1, ssem0, ssem1, dsem0, dsem1):
    c = lax.axis_index("c")
    s = lax.axis_index("s")
    sems = (sem0, sem1)
    ssems = (ssem0, ssem1)
    dsems = (dsem0, dsem1)

    def pipeline(h_ref, x_ref, x1_ref, w2_ref, h1_ref, es_ref, et_ref,
                 a2n_ref, newh_ref, h1n_ref):
        pltpu.sync_copy(x1_ref, x1tab)
        pltpu.sync_copy(h1_ref, h1tab)
        pltpu.sync_copy(a2n_ref, a2buf)

        nst = jnp.where(s < NSTEP - 19 * NS, 20, 19)
        lo = jnp.where(s < NSTEP - 19 * NS, 20 * s,
                       19 * s + (NSTEP - 19 * NS))

        def _step_base(k):
            return jnp.minimum((lo + k) * NB2, N - NB2)

        def ichunk(k, _):
            nb = _step_base(k)
            pltpu.sync_copy(x_ref.at[pl.ds(nb, NB2), :], nbuf)
            pltpu.sync_copy(w2_ref.at[pl.ds(nb, NB2)], w2buf)

            def ihalf(half, _):
                w2v = w2buf[pl.ds(half * L, L)]
                for e in range(L):
                    w2 = w2v[e]
                    r = half * L + e
                    for f in range(D // L):
                        nbuf[r, pl.ds(f * L, L)] = (
                            nbuf[r, pl.ds(f * L, L)] * w2)
                return 0
            lax.fori_loop(0, 2, ihalf, 0)
            pltpu.sync_copy(nbuf, agg_sp.at[pl.ds(nb, NB2), :])
            pltpu.sync_copy(w2buf, div_sp.at[pl.ds(nb, NB2)])
            return 0
        lax.fori_loop(0, nst, ichunk, 0)
        plsc.subcore_barrier()


        def compute_w1(j, b):
            def wgroup(g, _):
                gb = g * L
                t16 = tidx[j, pl.ds(gb, L)]
                s16 = sidx[j, pl.ds(gb, L)]
                h1t = plsc.load_gather(h1tab, [t16])
                x1s = plsc.load_gather(x1tab, [s16])
                w1buf[b, pl.ds(gb, L)] = jnp.exp(_leaky_v(x1s + h1t))
                return 0
            lax.fori_loop(0, EC // L, wgroup, 0)

        def issue_gather(j, b):
            pltpu.async_copy(h_ref.at[tidx.at[j]], rows.at[b], sems[b])

        def process(j, b):
            pltpu.make_async_copy(h_ref.at[tidx.at[j]], rows.at[b],
                                  sems[b]).wait()

            def egroup(g, _):
                gb = g * L
                w16 = w1buf[b, pl.ds(gb, L)]
                for e in range(L):
                    w = w16[e]
                    r = gb + e
                    for f in range(D // L):
                        rows[b, r, pl.ds(f * L, L)] = (
                            rows[b, r, pl.ds(f * L, L)] * w)
                return 0
            lax.fori_loop(0, EC // L, egroup, 0)
            pltpu.async_copy(rows.at[b], agg_sp.at[sidx.at[j]], ssems[b],
                             add=True)
            pltpu.async_copy(w1buf.at[b], div_sp.at[sidx.at[j]], dsems[b],
                             add=True)

        def bloop(blk, _):
            rowbase = (s + blk * NS) * BLKC
            pltpu.sync_copy(es_ref.at[pl.ds(rowbase, BLKC), :], sidx)
            pltpu.sync_copy(et_ref.at[pl.ds(rowbase, BLKC), :], tidx)
            issue_gather(0, 0)
            compute_w1(0, 0)

            def jpair(jp, _):
                for b in (0, 1):
                    j = jp * 2 + b

                    @pl.when(j + 1 < BLKC)
                    def _():
                        @pl.when(j >= 1)
                        def _():
                            pltpu.make_async_copy(
                                rows.at[1 - b], agg_sp.at[sidx.at[0]],
                                ssems[1 - b]).wait()
                            pltpu.make_async_copy(
                                w1buf.at[1 - b], div_sp.at[sidx.at[0]],
                                dsems[1 - b]).wait()
                        issue_gather(j + 1, 1 - b)
                        compute_w1(j + 1, 1 - b)

                    process(j, b)
                return 0
            lax.fori_loop(0, BLKC // 2, jpair, 0)
            for b in (0, 1):
                pltpu.make_async_copy(rows.at[b], agg_sp.at[sidx.at[0]],
                                      ssems[b]).wait()
                pltpu.make_async_copy(w1buf.at[b], div_sp.at[sidx.at[0]],
                                      dsems[b]).wait()
            return 0
        nbe = (NBLKT - s + NS - 1) // NS
        lax.fori_loop(0, nbe, bloop, 0)
        plsc.subcore_barrier()

        def nchunk(k, _):
            nb = _step_base(k)
            pltpu.sync_copy(agg_sp.at[pl.ds(nb, NB2), :], nbuf)
            pltpu.sync_copy(div_sp.at[pl.ds(nb, NB2)], divbuf)
            lane = lax.iota(jnp.int32, L)

            def nhalf(half, _):
                dvv = divbuf[pl.ds(half * L, L)]
                hv = jnp.zeros((L,), jnp.float32)
                rdvv = 1.0 / dvv
                for e in range(L):
                    rdv = rdvv[e]
                    r = half * L + e
                    acc = jnp.zeros((L,), jnp.float32)
                    for f in range(D // L):
                        v = nbuf[r, pl.ds(f * L, L)] * rdv
                        nh = jnp.where(v > 0, v, jnp.exp(v) - 1.0)
                        nbuf[r, pl.ds(f * L, L)] = nh
                        acc = acc + nh * a2buf[pl.ds(f * L, L)]
                    hv = jnp.where(lane == e, jnp.sum(acc), hv)
                h1nbuf[pl.ds(half * L, L)] = hv
                return 0
            lax.fori_loop(0, 2, nhalf, 0)
            pltpu.sync_copy(nbuf, newh_ref.at[pl.ds(nb, NB2), :])
            pltpu.sync_copy(h1nbuf, h1n_ref.at[pl.ds(nb, NB2)])
            return 0
        lax.fori_loop(0, nst, nchunk, 0)

    @pl.when(c == 0)
    def _():
        pipeline(hB, xA, x1ab, w2ab, h1B, sab, tab, a2ban, newhA, h1An)

    @pl.when(c == 1)
    def _():
        pipeline(hA, xB, x1ba, w2ba, h1A, sba, tba, a2abn, newhB, h1Bn)


def _run_hop(hA, hB, xA, xB, x1ab_i, w2ab_i, x1ba_i, w2ba_i, h1B, h1A,
             sab, tab, sba, tba, a2ab_next, a2ba_next):
    mesh = plsc.VectorSubcoreMesh(core_axis_name="c", subcore_axis_name="s",
                                  num_cores=NC, num_subcores=NS)
    f32 = jnp.float32
    i32 = jnp.int32
    out_type = [
        jax.ShapeDtypeStruct((N, D), f32),
        jax.ShapeDtypeStruct((N, D), f32),
        jax.ShapeDtypeStruct((N,), f32),
        jax.ShapeDtypeStruct((N,), f32),
    ]
    scratch = [
        pltpu.VMEM((N,), f32),
        pltpu.VMEM((N,), f32),
        pltpu.VMEM((D,), f32),
        pltpu.VMEM((BLKC, EC), i32),
        pltpu.VMEM((BLKC, EC), i32),
        pltpu.VMEM((2, EC, D), f32),
        pltpu.VMEM((2, EC), f32),
        pltpu.VMEM((NB2, D), f32),
        pltpu.VMEM((NB2,), f32),
        pltpu.VMEM((NB2,), f32),
        pltpu.VMEM((NB2,), f32),
        pltpu.VMEM_SHARED((N, D), f32),
        pltpu.VMEM_SHARED((N,), f32),
        pltpu.SemaphoreType.DMA,
        pltpu.SemaphoreType.DMA,
        pltpu.SemaphoreType.DMA,
        pltpu.SemaphoreType.DMA,
        pltpu.SemaphoreType.DMA,
        pltpu.SemaphoreType.DMA,
    ]
    run = pl.kernel(_hop_body, out_type=out_type, mesh=mesh,
                    scratch_types=scratch,
                    compiler_params=pltpu.CompilerParams(
                        needs_layout_passes=False))
    return run(hA, hB, xA, xB, x1ab_i, w2ab_i, x1ba_i, w2ba_i, h1B, h1A,
               sab, tab, sba, tba, a2ab_next, a2ba_next)


def kernel(x_A, x_B, edge_ab, edge_ba, W1_A, b1_A, W1_B, b1_B,
           a1_ab, a2_ab, a1_ba, a2_ba, W2, b2):
    (xA, xB, x1ab, w2ab, x1ba, w2ba, h1b0, h1a0) = _run_prologue(
        x_A, x_B, W1_A, b1_A, W1_B, b1_B, a1_ab, a2_ab, a1_ba, a2_ba)
    hA, hB = xA, xB
    h1B, h1A = h1b0.reshape(N), h1a0.reshape(N)
    sab, tab = edge_ab[0].reshape(E // EC, EC), edge_ab[1].reshape(E // EC, EC)
    sba, tba = edge_ba[0].reshape(E // EC, EC), edge_ba[1].reshape(E // EC, EC)
    for i in range(HOP):
        nxt = i + 1 if i + 1 < HOP else 0
        hA, hB, h1A, h1B = _run_hop(
            hA, hB, xA, xB, x1ab[:, i], w2ab[:, i], x1ba[:, i], w2ba[:, i],
            h1B, h1A, sab, tab, sba, tba, a2_ab[nxt], a2_ba[nxt])
    return _run_epilogue(hA, W2, b2)
